# Initial kernel scaffold; baseline (speedup 1.0000x reference)
import functools

import jax
import jax.numpy as jnp
from jax import lax
from jax.experimental import pallas as pl
from jax.experimental.pallas import tpu as pltpu
from jax.experimental.pallas import tpu_sc as plsc

N = 50000
D = 32
NUM_LAYERS_K = 3
E = 1600000

NC = 2
NS = 16
NW = NC * NS
N_PAD = 50048
PAD_NODE = N
BLK = 128
CB = 56
EB_TILE = 392
E_PAD = NW * EB_TILE * BLK
RPT = N_PAD // NS
ZROWS = RPT // 4

_mesh = plsc.VectorSubcoreMesh(core_axis_name="c", subcore_axis_name="s")


@functools.partial(
    pl.kernel,
    out_type=jax.ShapeDtypeStruct((NC, N_PAD, D), jnp.float32),
    mesh=_mesh,
    scratch_types=[
        pltpu.VMEM_SHARED((N_PAD, D), jnp.float32),
        pltpu.VMEM((CB, BLK), jnp.int32),
        pltpu.VMEM((CB, BLK), jnp.int32),
        pltpu.VMEM((BLK, D), jnp.float32),
        pltpu.VMEM((ZROWS, D), jnp.float32),
        pltpu.SemaphoreType.DMA,
    ],
)
def _spmv_sc(p_hbm, frm_hbm, to_hbm, out_hbm, accum, frm_v, to_v, rows_v, zb_v, sem):
    cid = lax.axis_index("c")
    sid = lax.axis_index("s")
    wid = cid * NS + sid

    zvec = jnp.zeros((16,), jnp.float32)

    def _zstore(i, _):
        zb_v[i, pl.ds(0, 16)] = zvec
        zb_v[i, pl.ds(16, 16)] = zvec
        return 0

    lax.fori_loop(0, ZROWS, _zstore, 0)

    def _zcopy(k, _):
        pltpu.sync_copy(zb_v, accum.at[pl.ds(sid * RPT + k * ZROWS, ZROWS)])
        return 0

    lax.fori_loop(0, RPT // ZROWS, _zcopy, 0)
    plsc.subcore_barrier()

    def _outer(ob, _):
        r0 = wid * EB_TILE + ob * CB
        pltpu.sync_copy(frm_hbm.at[pl.ds(r0, CB)], frm_v)
        pltpu.sync_copy(to_hbm.at[pl.ds(r0, CB)], to_v)

        def _inner(j, _):
            pltpu.async_copy(p_hbm.at[frm_v.at[j]], rows_v, sem).wait()
            pltpu.sync_copy(rows_v, accum.at[to_v.at[j]], add=True)
            return 0

        lax.fori_loop(0, CB, _inner, 0)
        return 0

    lax.fori_loop(0, EB_TILE // CB, _outer, 0)
    plsc.subcore_barrier()

    pltpu.sync_copy(
        accum.at[pl.ds(sid * RPT, RPT)],
        out_hbm.at[cid, pl.ds(sid * RPT, RPT)],
    )


_TCB = N_PAD // 8


def _dis_body(a0, a1, emb, dise, dis2e, p0):
    deg = a0[...][:, 0:1] + a1[...][:, 0:1]
    d = jnp.where(deg > 0.0, lax.rsqrt(jnp.maximum(deg, 1e-30)), 0.0)
    de = jnp.broadcast_to(d, (_TCB, D))
    dise[...] = de
    dis2e[...] = de * de
    p0[...] = de * emb[...]


_dis_tc = pl.pallas_call(
    _dis_body,
    grid=(8,),
    in_specs=[pl.BlockSpec((_TCB, D), lambda i: (i, 0))] * 3,
    out_specs=[pl.BlockSpec((_TCB, D), lambda i: (i, 0))] * 3,
    out_shape=[jax.ShapeDtypeStruct((N_PAD, D), jnp.float32)] * 3,
)


def _dense_body(scale, a0, a1, dise, dis2e, s_in, s_out, p_out):
    a = a0[...] + a1[...]
    s_out[...] = (s_in[...] + dise[...] * a) * scale
    p_out[...] = dis2e[...] * a


def _make_dense(scale):
    return pl.pallas_call(
        functools.partial(_dense_body, scale),
        grid=(8,),
        in_specs=[pl.BlockSpec((_TCB, D), lambda i: (i, 0))] * 5,
        out_specs=[pl.BlockSpec((_TCB, D), lambda i: (i, 0))] * 2,
        out_shape=[jax.ShapeDtypeStruct((N_PAD, D), jnp.float32)] * 2,
    )


_dense_mid = _make_dense(1.0)
_dense_last = _make_dense(1.0 / (NUM_LAYERS_K + 1))


def kernel(emb_weight, edge_index):
    frm = edge_index[0].astype(jnp.int32)
    to = edge_index[1].astype(jnp.int32)
    pad = jnp.full((E_PAD - E,), PAD_NODE, jnp.int32)
    frm_p = jnp.concatenate([frm, pad]).reshape(E_PAD // BLK, BLK)
    to_p = jnp.concatenate([to, pad]).reshape(E_PAD // BLK, BLK)

    emb_pad = jnp.zeros((N_PAD, D), jnp.float32).at[:N].set(emb_weight)
    ones_tbl = jnp.ones((N_PAD, D), jnp.float32)

    g = _spmv_sc(ones_tbl, frm_p, to_p)
    dise, dis2e, p = _dis_tc(g[0], g[1], emb_pad)

    s = emb_pad
    for layer in range(NUM_LAYERS_K):
        parts = _spmv_sc(p, frm_p, to_p)
        dense = _dense_last if layer == NUM_LAYERS_K - 1 else _dense_mid
        s, p = dense(parts[0], parts[1], dise, dis2e, s)

    return (emb_weight, s[:N])


# speedup vs baseline: 20.1055x; 20.1055x over previous
import functools

import jax
import jax.numpy as jnp
from jax import lax
from jax.experimental import pallas as pl
from jax.experimental.pallas import tpu as pltpu
from jax.experimental.pallas import tpu_sc as plsc

N = 50000
D = 32
NUM_LAYERS_K = 3
E = 1600000

NC = 2
NS = 16
NW = NC * NS
N_PAD = 50048
PAD_NODE = N
BLK = 128
CB = 56
EB_TILE = 392
E_PAD = NW * EB_TILE * BLK
RPT = N_PAD // NS

def _spmv_body(p_hbm, frm_hbm, to_hbm, out_hbm, accum, frm_v, to_v, rows_v, sem):
    cid = lax.axis_index("c")
    sid = lax.axis_index("s")
    wid = cid * NS + sid

    zvec = jnp.zeros((16,), jnp.float32)

    def _zstore(i, _):
        rows_v[i, pl.ds(0, 16)] = zvec
        rows_v[i, pl.ds(16, 16)] = zvec
        return 0

    lax.fori_loop(0, BLK, _zstore, 0)

    def _zcopy(k, _):
        pltpu.sync_copy(rows_v, accum.at[pl.ds(sid * RPT + k * BLK, BLK)])
        return 0

    nfull = RPT // BLK
    rem = RPT - nfull * BLK
    lax.fori_loop(0, nfull, _zcopy, 0)
    pltpu.sync_copy(
        rows_v.at[pl.ds(0, rem)],
        accum.at[pl.ds(sid * RPT + nfull * BLK, rem)],
    )
    plsc.subcore_barrier()

    def _outer(ob, _):
        r0 = wid * EB_TILE + ob * CB
        pltpu.sync_copy(frm_hbm.at[pl.ds(r0, CB)], frm_v)
        pltpu.sync_copy(to_hbm.at[pl.ds(r0, CB)], to_v)

        def _inner(j, _):
            pltpu.async_copy(p_hbm.at[frm_v.at[j]], rows_v, sem).wait()
            pltpu.sync_copy(rows_v, accum.at[to_v.at[j]], add=True)
            return 0

        lax.fori_loop(0, CB, _inner, 0)
        return 0

    lax.fori_loop(0, EB_TILE // CB, _outer, 0)
    plsc.subcore_barrier()

    pltpu.sync_copy(
        accum.at[pl.ds(sid * RPT, RPT)],
        out_hbm.at[cid, pl.ds(sid * RPT, RPT)],
    )


@functools.cache
def _get_spmv():
    mesh = plsc.VectorSubcoreMesh(
        core_axis_name="c", subcore_axis_name="s", num_cores=NC, num_subcores=NS
    )
    return pl.kernel(
        _spmv_body,
        out_type=jax.ShapeDtypeStruct((NC, N_PAD, D), jnp.float32),
        mesh=mesh,
        scratch_types=[
            pltpu.VMEM_SHARED((N_PAD, D), jnp.float32),
            pltpu.VMEM((CB, BLK), jnp.int32),
            pltpu.VMEM((CB, BLK), jnp.int32),
            pltpu.VMEM((BLK, D), jnp.float32),
            pltpu.SemaphoreType.DMA,
        ],
        compiler_params=pltpu.CompilerParams(use_tc_tiling_on_sc=False),
    )


_TCB = N_PAD // 8


def _dis_body(a0, a1, emb, dise, dis2e, p0):
    deg = a0[...][:, 0:1] + a1[...][:, 0:1]
    d = jnp.where(deg > 0.0, lax.rsqrt(jnp.maximum(deg, 1e-30)), 0.0)
    de = jnp.broadcast_to(d, (_TCB, D))
    dise[...] = de
    dis2e[...] = de * de
    p0[...] = de * emb[...]


_dis_tc = pl.pallas_call(
    _dis_body,
    grid=(8,),
    in_specs=[pl.BlockSpec((_TCB, D), lambda i: (i, 0))] * 3,
    out_specs=[pl.BlockSpec((_TCB, D), lambda i: (i, 0))] * 3,
    out_shape=[jax.ShapeDtypeStruct((N_PAD, D), jnp.float32)] * 3,
)


def _dense_body(scale, a0, a1, dise, dis2e, s_in, s_out, p_out):
    a = a0[...] + a1[...]
    s_out[...] = (s_in[...] + dise[...] * a) * scale
    p_out[...] = dis2e[...] * a


def _make_dense(scale):
    return pl.pallas_call(
        functools.partial(_dense_body, scale),
        grid=(8,),
        in_specs=[pl.BlockSpec((_TCB, D), lambda i: (i, 0))] * 5,
        out_specs=[pl.BlockSpec((_TCB, D), lambda i: (i, 0))] * 2,
        out_shape=[jax.ShapeDtypeStruct((N_PAD, D), jnp.float32)] * 2,
    )


_dense_mid = _make_dense(1.0)
_dense_last = _make_dense(1.0 / (NUM_LAYERS_K + 1))


def kernel(emb_weight, edge_index):
    frm = edge_index[0].astype(jnp.int32)
    to = edge_index[1].astype(jnp.int32)
    pad = jnp.full((E_PAD - E,), PAD_NODE, jnp.int32)
    frm_p = jnp.concatenate([frm, pad]).reshape(E_PAD // BLK, BLK)
    to_p = jnp.concatenate([to, pad]).reshape(E_PAD // BLK, BLK)

    emb_pad = jnp.zeros((N_PAD, D), jnp.float32).at[:N].set(emb_weight)
    ones_tbl = jnp.ones((N_PAD, D), jnp.float32)

    spmv = _get_spmv()
    g = spmv(ones_tbl, frm_p, to_p)
    dise, dis2e, p = _dis_tc(g[0], g[1], emb_pad)

    s = emb_pad
    for layer in range(NUM_LAYERS_K):
        parts = spmv(p, frm_p, to_p)
        dense = _dense_last if layer == NUM_LAYERS_K - 1 else _dense_mid
        s, p = dense(parts[0], parts[1], dise, dis2e, s)

    return (emb_weight, s[:N])


# speedup vs baseline: 34.2382x; 1.7029x over previous
import functools

import jax
import jax.numpy as jnp
from jax import lax
from jax.experimental import pallas as pl
from jax.experimental.pallas import tpu as pltpu
from jax.experimental.pallas import tpu_sc as plsc

N = 50000
D = 32
NUM_LAYERS_K = 3
E = 1600000

NC = 2
NS = 16
NW = NC * NS
N_PAD = 50048
PAD_NODE = N
BLK = 128
CB = 28
EB_TILE = 392
NB_OUT = EB_TILE // CB
E_PAD = NW * EB_TILE * BLK
RPT = N_PAD // NS
KB = 2
DEG_W = 16


def _zero_shared(buf2d, accum, row0, buf_rows):
    zvec = jnp.zeros((16,), jnp.float32)
    width = buf2d.shape[-1]
    npv = width // 16

    def _zstore(i, _):
        for v in range(npv):
            buf2d[i, pl.ds(v * 16, 16)] = zvec
        return 0

    lax.fori_loop(0, buf_rows, _zstore, 0)

    def _zcopy(k, _):
        pltpu.sync_copy(buf2d, accum.at[pl.ds(row0 + k * buf_rows, buf_rows)])
        return 0

    nfull = RPT // buf_rows
    rem = RPT - nfull * buf_rows
    lax.fori_loop(0, nfull, _zcopy, 0)
    if rem:
        pltpu.sync_copy(
            buf2d.at[pl.ds(0, rem)],
            accum.at[pl.ds(row0 + nfull * buf_rows, rem)],
        )


def _spmv_body(p_hbm, frm_hbm, to_hbm, out_hbm, accum, frm_v, to_v, rows_v,
               sem0, sem1):
    cid = lax.axis_index("c")
    sid = lax.axis_index("s")
    wid = cid * NS + sid

    _zero_shared(rows_v.at[0], accum, sid * RPT, BLK)
    plsc.subcore_barrier()

    sems = (sem0, sem1)
    nbatch = CB // KB

    def _fire(b):
        s = b % 2
        descs = []
        for i in range(KB):
            descs.append(
                pltpu.async_copy(
                    p_hbm.at[frm_v.at[b * KB + i]],
                    rows_v.at[s * KB + i],
                    sems[s],
                )
            )
        return descs

    def _outer(ob, _):
        r0 = wid * EB_TILE + ob * CB
        pltpu.sync_copy(frm_hbm.at[pl.ds(r0, CB)], frm_v)
        pltpu.sync_copy(to_hbm.at[pl.ds(r0, CB)], to_v)

        d_cur = _fire(0)
        for b in range(nbatch):
            d_nxt = _fire(b + 1) if b + 1 < nbatch else None
            for d in d_cur:
                d.wait()
            base = (b % 2) * KB
            for i in range(KB):
                pltpu.sync_copy(
                    rows_v.at[base + i],
                    accum.at[to_v.at[b * KB + i]],
                    add=True,
                )
            d_cur = d_nxt
        return 0

    lax.fori_loop(0, NB_OUT, _outer, 0)
    plsc.subcore_barrier()

    pltpu.sync_copy(
        accum.at[pl.ds(sid * RPT, RPT)],
        out_hbm.at[cid, pl.ds(sid * RPT, RPT)],
    )


def _deg_body(to_hbm, out_hbm, accum, to_v, buf_v):
    cid = lax.axis_index("c")
    sid = lax.axis_index("s")
    wid = cid * NS + sid

    _zero_shared(buf_v, accum, sid * RPT, BLK)
    plsc.subcore_barrier()

    onevec = jnp.ones((16,), jnp.float32)

    def _ostore(i, _):
        buf_v[i, pl.ds(0, 16)] = onevec
        return 0

    lax.fori_loop(0, BLK, _ostore, 0)

    def _outer(ob, _):
        r0 = wid * EB_TILE + ob * CB
        pltpu.sync_copy(to_hbm.at[pl.ds(r0, CB)], to_v)
        for j in range(CB):
            pltpu.sync_copy(buf_v, accum.at[to_v.at[j]], add=True)
        return 0

    lax.fori_loop(0, NB_OUT, _outer, 0)
    plsc.subcore_barrier()

    pltpu.sync_copy(
        accum.at[pl.ds(sid * RPT, RPT)],
        out_hbm.at[cid, pl.ds(sid * RPT, RPT)],
    )


@functools.cache
def _get_sc_kernels():
    mesh = plsc.VectorSubcoreMesh(
        core_axis_name="c", subcore_axis_name="s", num_cores=NC, num_subcores=NS
    )
    spmv = pl.kernel(
        _spmv_body,
        out_type=jax.ShapeDtypeStruct((NC, N_PAD, D), jnp.float32),
        mesh=mesh,
        scratch_types=[
            pltpu.VMEM_SHARED((N_PAD, D), jnp.float32),
            pltpu.VMEM((CB, BLK), jnp.int32),
            pltpu.VMEM((CB, BLK), jnp.int32),
            pltpu.VMEM((2 * KB, BLK, D), jnp.float32),
            pltpu.SemaphoreType.DMA,
            pltpu.SemaphoreType.DMA,
        ],
        compiler_params=pltpu.CompilerParams(use_tc_tiling_on_sc=False),
    )
    deg = pl.kernel(
        _deg_body,
        out_type=jax.ShapeDtypeStruct((NC, N_PAD, DEG_W), jnp.float32),
        mesh=mesh,
        scratch_types=[
            pltpu.VMEM_SHARED((N_PAD, DEG_W), jnp.float32),
            pltpu.VMEM((CB, BLK), jnp.int32),
            pltpu.VMEM((BLK, DEG_W), jnp.float32),
        ],
        compiler_params=pltpu.CompilerParams(use_tc_tiling_on_sc=False),
    )
    return spmv, deg


_TCB = N_PAD // 8


def _dis_body(a0, a1, emb, dise, dis2e, p0):
    deg = a0[...][:, 0:1] + a1[...][:, 0:1]
    d = jnp.where(deg > 0.0, lax.rsqrt(jnp.maximum(deg, 1e-30)), 0.0)
    de = jnp.broadcast_to(d, (_TCB, D))
    dise[...] = de
    dis2e[...] = de * de
    p0[...] = de * emb[...]


_dis_tc = pl.pallas_call(
    _dis_body,
    grid=(8,),
    in_specs=[
        pl.BlockSpec((_TCB, DEG_W), lambda i: (i, 0)),
        pl.BlockSpec((_TCB, DEG_W), lambda i: (i, 0)),
        pl.BlockSpec((_TCB, D), lambda i: (i, 0)),
    ],
    out_specs=[pl.BlockSpec((_TCB, D), lambda i: (i, 0))] * 3,
    out_shape=[jax.ShapeDtypeStruct((N_PAD, D), jnp.float32)] * 3,
)


def _dense_body(scale, a0, a1, dise, dis2e, s_in, s_out, p_out):
    a = a0[...] + a1[...]
    s_out[...] = (s_in[...] + dise[...] * a) * scale
    p_out[...] = dis2e[...] * a


def _make_dense(scale):
    return pl.pallas_call(
        functools.partial(_dense_body, scale),
        grid=(8,),
        in_specs=[pl.BlockSpec((_TCB, D), lambda i: (i, 0))] * 5,
        out_specs=[pl.BlockSpec((_TCB, D), lambda i: (i, 0))] * 2,
        out_shape=[jax.ShapeDtypeStruct((N_PAD, D), jnp.float32)] * 2,
    )


_dense_mid = _make_dense(1.0)
_dense_last = _make_dense(1.0 / (NUM_LAYERS_K + 1))


def kernel(emb_weight, edge_index):
    frm = edge_index[0].astype(jnp.int32)
    to = edge_index[1].astype(jnp.int32)
    pad = jnp.full((E_PAD - E,), PAD_NODE, jnp.int32)
    frm_p = jnp.concatenate([frm, pad]).reshape(E_PAD // BLK, BLK)
    to_p = jnp.concatenate([to, pad]).reshape(E_PAD // BLK, BLK)

    emb_pad = jnp.zeros((N_PAD, D), jnp.float32).at[:N].set(emb_weight)

    spmv, deg = _get_sc_kernels()
    g = deg(to_p)
    dise, dis2e, p = _dis_tc(g[0], g[1], emb_pad)

    s = emb_pad
    for layer in range(NUM_LAYERS_K):
        parts = spmv(p, frm_p, to_p)
        dense = _dense_last if layer == NUM_LAYERS_K - 1 else _dense_mid
        s, p = dense(parts[0], parts[1], dise, dis2e, s)

    return (emb_weight, s[:N])


# speedup vs baseline: 34.6363x; 1.0116x over previous
import functools

import jax
import jax.numpy as jnp
from jax import lax
from jax.experimental import pallas as pl
from jax.experimental.pallas import tpu as pltpu
from jax.experimental.pallas import tpu_sc as plsc

N = 50000
D = 32
NUM_LAYERS_K = 3
E = 1600000

NC = 2
NS = 16
NW = NC * NS
N_PAD = 50048
PAD_NODE = N
BLK = 128
CB = 28
EB_TILE = 392
NB_OUT = EB_TILE // CB
E_PAD = NW * EB_TILE * BLK
RPT = N_PAD // NS
KB = 2
DEG_W = 16


def _zero_shared(buf2d, accum, row0, buf_rows):
    zvec = jnp.zeros((16,), jnp.float32)
    width = buf2d.shape[-1]
    npv = width // 16

    def _zstore(i, _):
        for v in range(npv):
            buf2d[i, pl.ds(v * 16, 16)] = zvec
        return 0

    lax.fori_loop(0, buf_rows, _zstore, 0)

    def _zcopy(k, _):
        pltpu.sync_copy(buf2d, accum.at[pl.ds(row0 + k * buf_rows, buf_rows)])
        return 0

    nfull = RPT // buf_rows
    rem = RPT - nfull * buf_rows
    lax.fori_loop(0, nfull, _zcopy, 0)
    if rem:
        pltpu.sync_copy(
            buf2d.at[pl.ds(0, rem)],
            accum.at[pl.ds(row0 + nfull * buf_rows, rem)],
        )


def _spmv_body(p_hbm, frm_hbm, to_hbm, out_hbm, accum, frm_v, to_v, rows_v,
               sem0, sem1, ssem0, ssem1):
    cid = lax.axis_index("c")
    sid = lax.axis_index("s")
    wid = cid * NS + sid

    _zero_shared(rows_v.at[0], accum, sid * RPT, BLK)
    plsc.subcore_barrier()

    gsems = (sem0, sem1)
    ssems = (ssem0, ssem1)
    nbatch = CB // KB

    def _fire(b):
        s = b % 2
        descs = []
        for i in range(KB):
            descs.append(
                pltpu.async_copy(
                    p_hbm.at[frm_v.at[b * KB + i]],
                    rows_v.at[s * KB + i],
                    gsems[s],
                )
            )
        return descs

    def _outer(ob, _):
        r0 = wid * EB_TILE + ob * CB
        pltpu.sync_copy(frm_hbm.at[pl.ds(r0, CB)], frm_v)
        pltpu.sync_copy(to_hbm.at[pl.ds(r0, CB)], to_v)

        d_cur = _fire(0)
        pend = [None, None]
        for b in range(nbatch):
            s = b % 2
            if b + 1 < nbatch:
                if pend[1 - s] is not None:
                    for d in pend[1 - s]:
                        d.wait()
                    pend[1 - s] = None
                d_nxt = _fire(b + 1)
            else:
                d_nxt = None
            for d in d_cur:
                d.wait()
            sc = []
            for i in range(KB):
                sc.append(
                    pltpu.async_copy(
                        rows_v.at[s * KB + i],
                        accum.at[to_v.at[b * KB + i]],
                        ssems[s],
                        add=True,
                    )
                )
            pend[s] = sc
            d_cur = d_nxt
        for p in pend:
            if p is not None:
                for d in p:
                    d.wait()
        return 0

    lax.fori_loop(0, NB_OUT, _outer, 0)
    plsc.subcore_barrier()

    pltpu.sync_copy(
        accum.at[pl.ds(sid * RPT, RPT)],
        out_hbm.at[cid, pl.ds(sid * RPT, RPT)],
    )


def _deg_body(to_hbm, out_hbm, accum, to_v, buf_v):
    cid = lax.axis_index("c")
    sid = lax.axis_index("s")
    wid = cid * NS + sid

    _zero_shared(buf_v, accum, sid * RPT, BLK)
    plsc.subcore_barrier()

    onevec = jnp.ones((16,), jnp.float32)

    def _ostore(i, _):
        buf_v[i, pl.ds(0, 16)] = onevec
        return 0

    lax.fori_loop(0, BLK, _ostore, 0)

    def _outer(ob, _):
        r0 = wid * EB_TILE + ob * CB
        pltpu.sync_copy(to_hbm.at[pl.ds(r0, CB)], to_v)
        for j in range(CB):
            pltpu.sync_copy(buf_v, accum.at[to_v.at[j]], add=True)
        return 0

    lax.fori_loop(0, NB_OUT, _outer, 0)
    plsc.subcore_barrier()

    pltpu.sync_copy(
        accum.at[pl.ds(sid * RPT, RPT)],
        out_hbm.at[cid, pl.ds(sid * RPT, RPT)],
    )


@functools.cache
def _get_sc_kernels():
    mesh = plsc.VectorSubcoreMesh(
        core_axis_name="c", subcore_axis_name="s", num_cores=NC, num_subcores=NS
    )
    spmv = pl.kernel(
        _spmv_body,
        out_type=jax.ShapeDtypeStruct((NC, N_PAD, D), jnp.float32),
        mesh=mesh,
        scratch_types=[
            pltpu.VMEM_SHARED((N_PAD, D), jnp.float32),
            pltpu.VMEM((CB, BLK), jnp.int32),
            pltpu.VMEM((CB, BLK), jnp.int32),
            pltpu.VMEM((2 * KB, BLK, D), jnp.float32),
            pltpu.SemaphoreType.DMA,
            pltpu.SemaphoreType.DMA,
            pltpu.SemaphoreType.DMA,
            pltpu.SemaphoreType.DMA,
        ],
        compiler_params=pltpu.CompilerParams(use_tc_tiling_on_sc=False),
    )
    deg = pl.kernel(
        _deg_body,
        out_type=jax.ShapeDtypeStruct((NC, N_PAD, DEG_W), jnp.float32),
        mesh=mesh,
        scratch_types=[
            pltpu.VMEM_SHARED((N_PAD, DEG_W), jnp.float32),
            pltpu.VMEM((CB, BLK), jnp.int32),
            pltpu.VMEM((BLK, DEG_W), jnp.float32),
        ],
        compiler_params=pltpu.CompilerParams(use_tc_tiling_on_sc=False),
    )
    return spmv, deg


_TCB = N_PAD // 8


def _dis_body(a0, a1, emb, dise, dis2e, p0):
    deg = a0[...][:, 0:1] + a1[...][:, 0:1]
    d = jnp.where(deg > 0.0, lax.rsqrt(jnp.maximum(deg, 1e-30)), 0.0)
    de = jnp.broadcast_to(d, (_TCB, D))
    dise[...] = de
    dis2e[...] = de * de
    p0[...] = de * emb[...]


_dis_tc = pl.pallas_call(
    _dis_body,
    grid=(8,),
    in_specs=[
        pl.BlockSpec((_TCB, DEG_W), lambda i: (i, 0)),
        pl.BlockSpec((_TCB, DEG_W), lambda i: (i, 0)),
        pl.BlockSpec((_TCB, D), lambda i: (i, 0)),
    ],
    out_specs=[pl.BlockSpec((_TCB, D), lambda i: (i, 0))] * 3,
    out_shape=[jax.ShapeDtypeStruct((N_PAD, D), jnp.float32)] * 3,
)


def _dense_body(scale, a0, a1, dise, dis2e, s_in, s_out, p_out):
    a = a0[...] + a1[...]
    s_out[...] = (s_in[...] + dise[...] * a) * scale
    p_out[...] = dis2e[...] * a


def _make_dense(scale):
    return pl.pallas_call(
        functools.partial(_dense_body, scale),
        grid=(8,),
        in_specs=[pl.BlockSpec((_TCB, D), lambda i: (i, 0))] * 5,
        out_specs=[pl.BlockSpec((_TCB, D), lambda i: (i, 0))] * 2,
        out_shape=[jax.ShapeDtypeStruct((N_PAD, D), jnp.float32)] * 2,
    )


_dense_mid = _make_dense(1.0)
_dense_last = _make_dense(1.0 / (NUM_LAYERS_K + 1))


def kernel(emb_weight, edge_index):
    frm = edge_index[0].astype(jnp.int32)
    to = edge_index[1].astype(jnp.int32)
    pad = jnp.full((E_PAD - E,), PAD_NODE, jnp.int32)
    frm_p = jnp.concatenate([frm, pad]).reshape(E_PAD // BLK, BLK)
    to_p = jnp.concatenate([to, pad]).reshape(E_PAD // BLK, BLK)

    emb_pad = jnp.zeros((N_PAD, D), jnp.float32).at[:N].set(emb_weight)

    spmv, deg = _get_sc_kernels()
    g = deg(to_p)
    dise, dis2e, p = _dis_tc(g[0], g[1], emb_pad)

    s = emb_pad
    for layer in range(NUM_LAYERS_K):
        parts = spmv(p, frm_p, to_p)
        dense = _dense_last if layer == NUM_LAYERS_K - 1 else _dense_mid
        s, p = dense(parts[0], parts[1], dise, dis2e, s)

    return (emb_weight, s[:N])


# speedup vs baseline: 34.9409x; 1.0088x over previous
import functools

import jax
import jax.numpy as jnp
from jax import lax
from jax.experimental import pallas as pl
from jax.experimental.pallas import tpu as pltpu
from jax.experimental.pallas import tpu_sc as plsc

N = 50000
D = 32
NUM_LAYERS_K = 3
E = 1600000

NC = 2
NS = 16
NW = NC * NS
N_PAD = 50048
PAD_NODE = N
BLK = 128
SBLK = 256
CB = 14
EB_TILE = 196
NB_OUT = EB_TILE // CB
E_PAD = NW * EB_TILE * SBLK
RPT = N_PAD // NS
DEG_W = 16


def _zero_shared(buf2d, accum, row0, buf_rows):
    zvec = jnp.zeros((16,), jnp.float32)
    width = buf2d.shape[-1]
    npv = width // 16

    def _zstore(i, _):
        for v in range(npv):
            buf2d[i, pl.ds(v * 16, 16)] = zvec
        return 0

    lax.fori_loop(0, buf_rows, _zstore, 0)

    def _zcopy(k, _):
        pltpu.sync_copy(buf2d, accum.at[pl.ds(row0 + k * buf_rows, buf_rows)])
        return 0

    nfull = RPT // buf_rows
    rem = RPT - nfull * buf_rows
    lax.fori_loop(0, nfull, _zcopy, 0)
    if rem:
        pltpu.sync_copy(
            buf2d.at[pl.ds(0, rem)],
            accum.at[pl.ds(row0 + nfull * buf_rows, rem)],
        )


def _spmv_body(p_hbm, frm_hbm, to_hbm, out_hbm, accum, frm_v, to_v, rows_v,
               sem0, sem1, ssem0, ssem1):
    cid = lax.axis_index("c")
    sid = lax.axis_index("s")
    wid = cid * NS + sid

    _zero_shared(rows_v.at[0], accum, sid * RPT, SBLK)
    plsc.subcore_barrier()

    gsems = (sem0, sem1)
    ssems = (ssem0, ssem1)
    nbatch = CB

    def _fire(b):
        s = b % 2
        return [
            pltpu.async_copy(
                p_hbm.at[frm_v.at[b]],
                rows_v.at[s],
                gsems[s],
            )
        ]

    def _outer(ob, _):
        r0 = wid * EB_TILE + ob * CB
        pltpu.sync_copy(frm_hbm.at[pl.ds(r0, CB)], frm_v)
        pltpu.sync_copy(to_hbm.at[pl.ds(r0, CB)], to_v)

        d_cur = _fire(0)
        pend = [None, None]
        for b in range(nbatch):
            s = b % 2
            if b + 1 < nbatch:
                if pend[1 - s] is not None:
                    for d in pend[1 - s]:
                        d.wait()
                    pend[1 - s] = None
                d_nxt = _fire(b + 1)
            else:
                d_nxt = None
            for d in d_cur:
                d.wait()
            pend[s] = [
                pltpu.async_copy(
                    rows_v.at[s],
                    accum.at[to_v.at[b]],
                    ssems[s],
                    add=True,
                )
            ]
            d_cur = d_nxt
        for p in pend:
            if p is not None:
                for d in p:
                    d.wait()
        return 0

    lax.fori_loop(0, NB_OUT, _outer, 0)
    plsc.subcore_barrier()

    pltpu.sync_copy(
        accum.at[pl.ds(sid * RPT, RPT)],
        out_hbm.at[cid, pl.ds(sid * RPT, RPT)],
    )


def _deg_body(to_hbm, out_hbm, accum, to_v, buf_v):
    cid = lax.axis_index("c")
    sid = lax.axis_index("s")
    wid = cid * NS + sid

    _zero_shared(buf_v, accum, sid * RPT, SBLK)
    plsc.subcore_barrier()

    onevec = jnp.ones((16,), jnp.float32)

    def _ostore(i, _):
        buf_v[i, pl.ds(0, 16)] = onevec
        return 0

    lax.fori_loop(0, SBLK, _ostore, 0)

    def _outer(ob, _):
        r0 = wid * EB_TILE + ob * CB
        pltpu.sync_copy(to_hbm.at[pl.ds(r0, CB)], to_v)
        for j in range(CB):
            pltpu.sync_copy(buf_v, accum.at[to_v.at[j]], add=True)
        return 0

    lax.fori_loop(0, NB_OUT, _outer, 0)
    plsc.subcore_barrier()

    pltpu.sync_copy(
        accum.at[pl.ds(sid * RPT, RPT)],
        out_hbm.at[cid, pl.ds(sid * RPT, RPT)],
    )


@functools.cache
def _get_sc_kernels():
    mesh = plsc.VectorSubcoreMesh(
        core_axis_name="c", subcore_axis_name="s", num_cores=NC, num_subcores=NS
    )
    spmv = pl.kernel(
        _spmv_body,
        out_type=jax.ShapeDtypeStruct((NC, N_PAD, D), jnp.float32),
        mesh=mesh,
        scratch_types=[
            pltpu.VMEM_SHARED((N_PAD, D), jnp.float32),
            pltpu.VMEM((CB, SBLK), jnp.int32),
            pltpu.VMEM((CB, SBLK), jnp.int32),
            pltpu.VMEM((2, SBLK, D), jnp.float32),
            pltpu.SemaphoreType.DMA,
            pltpu.SemaphoreType.DMA,
            pltpu.SemaphoreType.DMA,
            pltpu.SemaphoreType.DMA,
        ],
        compiler_params=pltpu.CompilerParams(use_tc_tiling_on_sc=False),
    )
    deg = pl.kernel(
        _deg_body,
        out_type=jax.ShapeDtypeStruct((NC, N_PAD, DEG_W), jnp.float32),
        mesh=mesh,
        scratch_types=[
            pltpu.VMEM_SHARED((N_PAD, DEG_W), jnp.float32),
            pltpu.VMEM((CB, SBLK), jnp.int32),
            pltpu.VMEM((SBLK, DEG_W), jnp.float32),
        ],
        compiler_params=pltpu.CompilerParams(use_tc_tiling_on_sc=False),
    )
    return spmv, deg


_TCB = N_PAD // 8


def _dis_body(a0, a1, emb, dise, dis2e, p0):
    deg = a0[...][:, 0:1] + a1[...][:, 0:1]
    d = jnp.where(deg > 0.0, lax.rsqrt(jnp.maximum(deg, 1e-30)), 0.0)
    de = jnp.broadcast_to(d, (_TCB, D))
    dise[...] = de
    dis2e[...] = de * de
    p0[...] = de * emb[...]


_dis_tc = pl.pallas_call(
    _dis_body,
    grid=(8,),
    in_specs=[
        pl.BlockSpec((_TCB, DEG_W), lambda i: (i, 0)),
        pl.BlockSpec((_TCB, DEG_W), lambda i: (i, 0)),
        pl.BlockSpec((_TCB, D), lambda i: (i, 0)),
    ],
    out_specs=[pl.BlockSpec((_TCB, D), lambda i: (i, 0))] * 3,
    out_shape=[jax.ShapeDtypeStruct((N_PAD, D), jnp.float32)] * 3,
)


def _dense_body(scale, a0, a1, dise, dis2e, s_in, s_out, p_out):
    a = a0[...] + a1[...]
    s_out[...] = (s_in[...] + dise[...] * a) * scale
    p_out[...] = dis2e[...] * a


def _make_dense(scale):
    return pl.pallas_call(
        functools.partial(_dense_body, scale),
        grid=(8,),
        in_specs=[pl.BlockSpec((_TCB, D), lambda i: (i, 0))] * 5,
        out_specs=[pl.BlockSpec((_TCB, D), lambda i: (i, 0))] * 2,
        out_shape=[jax.ShapeDtypeStruct((N_PAD, D), jnp.float32)] * 2,
    )


_dense_mid = _make_dense(1.0)
_dense_last = _make_dense(1.0 / (NUM_LAYERS_K + 1))


def kernel(emb_weight, edge_index):
    frm = edge_index[0].astype(jnp.int32)
    to = edge_index[1].astype(jnp.int32)
    pad = jnp.full((E_PAD - E,), PAD_NODE, jnp.int32)
    frm_p = jnp.concatenate([frm, pad]).reshape(E_PAD // SBLK, SBLK)
    to_p = jnp.concatenate([to, pad]).reshape(E_PAD // SBLK, SBLK)

    emb_pad = jnp.zeros((N_PAD, D), jnp.float32).at[:N].set(emb_weight)

    spmv, deg = _get_sc_kernels()
    g = deg(to_p)
    dise, dis2e, p = _dis_tc(g[0], g[1], emb_pad)

    s = emb_pad
    for layer in range(NUM_LAYERS_K):
        parts = spmv(p, frm_p, to_p)
        dense = _dense_last if layer == NUM_LAYERS_K - 1 else _dense_mid
        s, p = dense(parts[0], parts[1], dise, dis2e, s)

    return (emb_weight, s[:N])


# speedup vs baseline: 35.0854x; 1.0041x over previous
import functools

import jax
import jax.numpy as jnp
from jax import lax
from jax.experimental import pallas as pl
from jax.experimental.pallas import tpu as pltpu
from jax.experimental.pallas import tpu_sc as plsc

N = 50000
D = 32
NUM_LAYERS_K = 3
E = 1600000

NC = 2
NS = 16
NW = NC * NS
N_PAD = 50048
PAD_NODE = N
BLK = 128
SBLK = 256
CB = 14
EB_TILE = 196
NB_OUT = EB_TILE // CB
E_PAD = NW * EB_TILE * SBLK
RPT = N_PAD // NS
DEG_W = 16


def _zero_shared(buf2d, accum, row0, buf_rows):
    zvec = jnp.zeros((16,), jnp.float32)
    width = buf2d.shape[-1]
    npv = width // 16

    def _zstore(i, _):
        for v in range(npv):
            buf2d[i, pl.ds(v * 16, 16)] = zvec
        return 0

    lax.fori_loop(0, buf_rows, _zstore, 0)

    def _zcopy(k, _):
        pltpu.sync_copy(buf2d, accum.at[pl.ds(row0 + k * buf_rows, buf_rows)])
        return 0

    nfull = RPT // buf_rows
    rem = RPT - nfull * buf_rows
    lax.fori_loop(0, nfull, _zcopy, 0)
    if rem:
        pltpu.sync_copy(
            buf2d.at[pl.ds(0, rem)],
            accum.at[pl.ds(row0 + nfull * buf_rows, rem)],
        )


def _spmv_body(p_hbm, frm_hbm, to_hbm, out_hbm, accum, frm_v, to_v, rows_v,
               sem0, sem1, ssem0, ssem1):
    cid = lax.axis_index("c")
    sid = lax.axis_index("s")
    wid = cid * NS + sid

    _zero_shared(rows_v.at[0], accum, sid * RPT, SBLK)
    plsc.subcore_barrier()

    gsems = (sem0, sem1)
    ssems = (ssem0, ssem1)
    nbatch = CB

    def _fire(b):
        s = b % 2
        return [
            pltpu.async_copy(
                p_hbm.at[frm_v.at[b, pl.ds(i * 128, 128)]],
                rows_v.at[s, pl.ds(i * 128, 128)],
                gsems[s],
            )
            for i in range(2)
        ]

    def _outer(ob, _):
        r0 = wid * EB_TILE + ob * CB
        pltpu.sync_copy(frm_hbm.at[pl.ds(r0, CB)], frm_v)
        pltpu.sync_copy(to_hbm.at[pl.ds(r0, CB)], to_v)

        d_cur = _fire(0)
        pend = [None, None]
        for b in range(nbatch):
            s = b % 2
            if b + 1 < nbatch:
                if pend[1 - s] is not None:
                    for d in pend[1 - s]:
                        d.wait()
                    pend[1 - s] = None
                d_nxt = _fire(b + 1)
            else:
                d_nxt = None
            for d in d_cur:
                d.wait()
            pend[s] = [
                pltpu.async_copy(
                    rows_v.at[s],
                    accum.at[to_v.at[b]],
                    ssems[s],
                    add=True,
                )
            ]
            d_cur = d_nxt
        for p in pend:
            if p is not None:
                for d in p:
                    d.wait()
        return 0

    lax.fori_loop(0, NB_OUT, _outer, 0)
    plsc.subcore_barrier()

    pltpu.sync_copy(
        accum.at[pl.ds(sid * RPT, RPT)],
        out_hbm.at[cid, pl.ds(sid * RPT, RPT)],
    )


def _deg_body(to_hbm, out_hbm, accum, to_v, buf_v, dsem):
    cid = lax.axis_index("c")
    sid = lax.axis_index("s")
    wid = cid * NS + sid

    _zero_shared(buf_v, accum, sid * RPT, SBLK)
    plsc.subcore_barrier()

    onevec = jnp.ones((16,), jnp.float32)

    def _ostore(i, _):
        buf_v[i, pl.ds(0, 16)] = onevec
        return 0

    lax.fori_loop(0, SBLK, _ostore, 0)

    def _outer(ob, _):
        r0 = wid * EB_TILE + ob * CB
        pltpu.sync_copy(to_hbm.at[pl.ds(r0, CB)], to_v)
        descs = []
        for j in range(CB):
            descs.append(
                pltpu.async_copy(buf_v, accum.at[to_v.at[j]], dsem, add=True)
            )
        for d in descs:
            d.wait()
        return 0

    lax.fori_loop(0, NB_OUT, _outer, 0)
    plsc.subcore_barrier()

    pltpu.sync_copy(
        accum.at[pl.ds(sid * RPT, RPT)],
        out_hbm.at[cid, pl.ds(sid * RPT, RPT)],
    )


@functools.cache
def _get_sc_kernels():
    mesh = plsc.VectorSubcoreMesh(
        core_axis_name="c", subcore_axis_name="s", num_cores=NC, num_subcores=NS
    )
    spmv = pl.kernel(
        _spmv_body,
        out_type=jax.ShapeDtypeStruct((NC, N_PAD, D), jnp.float32),
        mesh=mesh,
        scratch_types=[
            pltpu.VMEM_SHARED((N_PAD, D), jnp.float32),
            pltpu.VMEM((CB, SBLK), jnp.int32),
            pltpu.VMEM((CB, SBLK), jnp.int32),
            pltpu.VMEM((2, SBLK, D), jnp.float32),
            pltpu.SemaphoreType.DMA,
            pltpu.SemaphoreType.DMA,
            pltpu.SemaphoreType.DMA,
            pltpu.SemaphoreType.DMA,
        ],
        compiler_params=pltpu.CompilerParams(use_tc_tiling_on_sc=False),
    )
    deg = pl.kernel(
        _deg_body,
        out_type=jax.ShapeDtypeStruct((NC, N_PAD, DEG_W), jnp.float32),
        mesh=mesh,
        scratch_types=[
            pltpu.VMEM_SHARED((N_PAD, DEG_W), jnp.float32),
            pltpu.VMEM((CB, SBLK), jnp.int32),
            pltpu.VMEM((SBLK, DEG_W), jnp.float32),
            pltpu.SemaphoreType.DMA,
        ],
        compiler_params=pltpu.CompilerParams(use_tc_tiling_on_sc=False),
    )
    return spmv, deg


_TCB = N_PAD // 8


def _dis_body(a0, a1, emb, dise, dis2e, p0):
    deg = a0[...][:, 0:1] + a1[...][:, 0:1]
    d = jnp.where(deg > 0.0, lax.rsqrt(jnp.maximum(deg, 1e-30)), 0.0)
    de = jnp.broadcast_to(d, (_TCB, D))
    dise[...] = de
    dis2e[...] = de * de
    p0[...] = de * emb[...]


_dis_tc = pl.pallas_call(
    _dis_body,
    grid=(8,),
    in_specs=[
        pl.BlockSpec((_TCB, DEG_W), lambda i: (i, 0)),
        pl.BlockSpec((_TCB, DEG_W), lambda i: (i, 0)),
        pl.BlockSpec((_TCB, D), lambda i: (i, 0)),
    ],
    out_specs=[pl.BlockSpec((_TCB, D), lambda i: (i, 0))] * 3,
    out_shape=[jax.ShapeDtypeStruct((N_PAD, D), jnp.float32)] * 3,
)


def _dense_body(scale, a0, a1, dise, dis2e, s_in, s_out, p_out):
    a = a0[...] + a1[...]
    s_out[...] = (s_in[...] + dise[...] * a) * scale
    p_out[...] = dis2e[...] * a


def _make_dense(scale):
    return pl.pallas_call(
        functools.partial(_dense_body, scale),
        grid=(8,),
        in_specs=[pl.BlockSpec((_TCB, D), lambda i: (i, 0))] * 5,
        out_specs=[pl.BlockSpec((_TCB, D), lambda i: (i, 0))] * 2,
        out_shape=[jax.ShapeDtypeStruct((N_PAD, D), jnp.float32)] * 2,
    )


_dense_mid = _make_dense(1.0)
_dense_last = _make_dense(1.0 / (NUM_LAYERS_K + 1))


def kernel(emb_weight, edge_index):
    frm = edge_index[0].astype(jnp.int32)
    to = edge_index[1].astype(jnp.int32)
    pad = jnp.full((E_PAD - E,), PAD_NODE, jnp.int32)
    frm_p = jnp.concatenate([frm, pad]).reshape(E_PAD // SBLK, SBLK)
    to_p = jnp.concatenate([to, pad]).reshape(E_PAD // SBLK, SBLK)

    emb_pad = jnp.zeros((N_PAD, D), jnp.float32).at[:N].set(emb_weight)

    spmv, deg = _get_sc_kernels()
    g = deg(to_p)
    dise, dis2e, p = _dis_tc(g[0], g[1], emb_pad)

    s = emb_pad
    for layer in range(NUM_LAYERS_K):
        parts = spmv(p, frm_p, to_p)
        dense = _dense_last if layer == NUM_LAYERS_K - 1 else _dense_mid
        s, p = dense(parts[0], parts[1], dise, dis2e, s)

    return (emb_weight, s[:N])


# speedup vs baseline: 41.0942x; 1.1713x over previous
import functools

import jax
import jax.numpy as jnp
from jax import lax
from jax.experimental import pallas as pl
from jax.experimental.pallas import tpu as pltpu
from jax.experimental.pallas import tpu_sc as plsc

N = 50000
D = 32
NUM_LAYERS_K = 3
E = 1600000

NC = 2
NS = 16
NW = NC * NS
N_PAD = 50048
PAD_NODE = N
BLK = 128
SBLK = 256
CB = 14
EB_TILE = 196
NB_OUT = EB_TILE // CB
E_PAD = NW * EB_TILE * SBLK
RPT = N_PAD // NS
DEG_W = 16


def _zero_shared(buf2d, accum, row0, buf_rows):
    lanes = 32 if buf2d.dtype == jnp.bfloat16 else 16
    zvec = jnp.zeros((lanes,), buf2d.dtype)
    width = buf2d.shape[-1]
    npv = width // lanes

    def _zstore(i, _):
        for v in range(npv):
            buf2d[i, pl.ds(v * lanes, lanes)] = zvec
        return 0

    lax.fori_loop(0, buf_rows, _zstore, 0)

    def _zcopy(k, _):
        pltpu.sync_copy(buf2d, accum.at[pl.ds(row0 + k * buf_rows, buf_rows)])
        return 0

    nfull = RPT // buf_rows
    rem = RPT - nfull * buf_rows
    lax.fori_loop(0, nfull, _zcopy, 0)
    if rem:
        pltpu.sync_copy(
            buf2d.at[pl.ds(0, rem)],
            accum.at[pl.ds(row0 + nfull * buf_rows, rem)],
        )


def _spmv_body(p_hbm, frm_hbm, to_hbm, out_hbm, accum, frm_v, to_v, rows_v,
               sem0, sem1, ssem0, ssem1):
    cid = lax.axis_index("c")
    sid = lax.axis_index("s")
    wid = cid * NS + sid

    _zero_shared(rows_v.at[0], accum, sid * RPT, SBLK)
    plsc.subcore_barrier()

    gsems = (sem0, sem1)
    ssems = (ssem0, ssem1)
    nbatch = CB

    def _fire(b):
        s = b % 2
        return [
            pltpu.async_copy(
                p_hbm.at[frm_v.at[b, pl.ds(i * 128, 128)]],
                rows_v.at[s, pl.ds(i * 128, 128)],
                gsems[s],
            )
            for i in range(2)
        ]

    def _outer(ob, _):
        r0 = wid * EB_TILE + ob * CB
        pltpu.sync_copy(frm_hbm.at[pl.ds(r0, CB)], frm_v)
        pltpu.sync_copy(to_hbm.at[pl.ds(r0, CB)], to_v)

        d_cur = _fire(0)
        pend = [None, None]
        for b in range(nbatch):
            s = b % 2
            if b + 1 < nbatch:
                if pend[1 - s] is not None:
                    for d in pend[1 - s]:
                        d.wait()
                    pend[1 - s] = None
                d_nxt = _fire(b + 1)
            else:
                d_nxt = None
            for d in d_cur:
                d.wait()
            pend[s] = [
                pltpu.async_copy(
                    rows_v.at[s],
                    accum.at[to_v.at[b]],
                    ssems[s],
                    add=True,
                )
            ]
            d_cur = d_nxt
        for p in pend:
            if p is not None:
                for d in p:
                    d.wait()
        return 0

    lax.fori_loop(0, NB_OUT, _outer, 0)
    plsc.subcore_barrier()

    pltpu.sync_copy(
        accum.at[pl.ds(sid * RPT, RPT)],
        out_hbm.at[cid, pl.ds(sid * RPT, RPT)],
    )


def _deg_body(to_hbm, out_hbm, accum, to_v, buf_v, dsem):
    cid = lax.axis_index("c")
    sid = lax.axis_index("s")
    wid = cid * NS + sid

    _zero_shared(buf_v, accum, sid * RPT, SBLK)
    plsc.subcore_barrier()

    onevec = jnp.ones((16,), jnp.float32)

    def _ostore(i, _):
        buf_v[i, pl.ds(0, 16)] = onevec
        return 0

    lax.fori_loop(0, SBLK, _ostore, 0)

    def _outer(ob, _):
        r0 = wid * EB_TILE + ob * CB
        pltpu.sync_copy(to_hbm.at[pl.ds(r0, CB)], to_v)
        descs = []
        for j in range(CB):
            descs.append(
                pltpu.async_copy(buf_v, accum.at[to_v.at[j]], dsem, add=True)
            )
        for d in descs:
            d.wait()
        return 0

    lax.fori_loop(0, NB_OUT, _outer, 0)
    plsc.subcore_barrier()

    pltpu.sync_copy(
        accum.at[pl.ds(sid * RPT, RPT)],
        out_hbm.at[cid, pl.ds(sid * RPT, RPT)],
    )


@functools.cache
def _get_sc_kernels():
    mesh = plsc.VectorSubcoreMesh(
        core_axis_name="c", subcore_axis_name="s", num_cores=NC, num_subcores=NS
    )
    spmv = pl.kernel(
        _spmv_body,
        out_type=jax.ShapeDtypeStruct((NC, N_PAD, D), jnp.bfloat16),
        mesh=mesh,
        scratch_types=[
            pltpu.VMEM_SHARED((N_PAD, D), jnp.bfloat16),
            pltpu.VMEM((CB, SBLK), jnp.int32),
            pltpu.VMEM((CB, SBLK), jnp.int32),
            pltpu.VMEM((2, SBLK, D), jnp.bfloat16),
            pltpu.SemaphoreType.DMA,
            pltpu.SemaphoreType.DMA,
            pltpu.SemaphoreType.DMA,
            pltpu.SemaphoreType.DMA,
        ],
        compiler_params=pltpu.CompilerParams(use_tc_tiling_on_sc=False),
    )
    deg = pl.kernel(
        _deg_body,
        out_type=jax.ShapeDtypeStruct((NC, N_PAD, DEG_W), jnp.float32),
        mesh=mesh,
        scratch_types=[
            pltpu.VMEM_SHARED((N_PAD, DEG_W), jnp.float32),
            pltpu.VMEM((CB, SBLK), jnp.int32),
            pltpu.VMEM((SBLK, DEG_W), jnp.float32),
            pltpu.SemaphoreType.DMA,
        ],
        compiler_params=pltpu.CompilerParams(use_tc_tiling_on_sc=False),
    )
    return spmv, deg


_TCB = N_PAD // 8


def _dis_body(a0, a1, emb, dise, dis2e, p0):
    deg = a0[...][:, 0:1] + a1[...][:, 0:1]
    d = jnp.where(deg > 0.0, lax.rsqrt(jnp.maximum(deg, 1e-30)), 0.0)
    de = jnp.broadcast_to(d, (_TCB, D))
    dise[...] = de
    dis2e[...] = de * de
    p0[...] = (de * emb[...]).astype(jnp.bfloat16)


_dis_tc = pl.pallas_call(
    _dis_body,
    grid=(8,),
    in_specs=[
        pl.BlockSpec((_TCB, DEG_W), lambda i: (i, 0)),
        pl.BlockSpec((_TCB, DEG_W), lambda i: (i, 0)),
        pl.BlockSpec((_TCB, D), lambda i: (i, 0)),
    ],
    out_specs=[pl.BlockSpec((_TCB, D), lambda i: (i, 0))] * 3,
    out_shape=[
        jax.ShapeDtypeStruct((N_PAD, D), jnp.float32),
        jax.ShapeDtypeStruct((N_PAD, D), jnp.float32),
        jax.ShapeDtypeStruct((N_PAD, D), jnp.bfloat16),
    ],
)


def _dense_body(scale, a0, a1, dise, dis2e, s_in, s_out, p_out):
    a = a0[...].astype(jnp.float32) + a1[...].astype(jnp.float32)
    s_out[...] = (s_in[...] + dise[...] * a) * scale
    p_out[...] = (dis2e[...] * a).astype(jnp.bfloat16)


def _make_dense(scale):
    return pl.pallas_call(
        functools.partial(_dense_body, scale),
        grid=(8,),
        in_specs=[pl.BlockSpec((_TCB, D), lambda i: (i, 0))] * 5,
        out_specs=[pl.BlockSpec((_TCB, D), lambda i: (i, 0))] * 2,
        out_shape=[
            jax.ShapeDtypeStruct((N_PAD, D), jnp.float32),
            jax.ShapeDtypeStruct((N_PAD, D), jnp.bfloat16),
        ],
    )


_dense_mid = _make_dense(1.0)
_dense_last = _make_dense(1.0 / (NUM_LAYERS_K + 1))


def kernel(emb_weight, edge_index):
    frm = edge_index[0].astype(jnp.int32)
    to = edge_index[1].astype(jnp.int32)
    pad = jnp.full((E_PAD - E,), PAD_NODE, jnp.int32)
    frm_p = jnp.concatenate([frm, pad]).reshape(E_PAD // SBLK, SBLK)
    to_p = jnp.concatenate([to, pad]).reshape(E_PAD // SBLK, SBLK)

    emb_pad = jnp.zeros((N_PAD, D), jnp.float32).at[:N].set(emb_weight)

    spmv, deg = _get_sc_kernels()
    g = deg(to_p)
    dise, dis2e, p = _dis_tc(g[0], g[1], emb_pad)

    s = emb_pad
    for layer in range(NUM_LAYERS_K):
        parts = spmv(p, frm_p, to_p)
        dense = _dense_last if layer == NUM_LAYERS_K - 1 else _dense_mid
        s, p = dense(parts[0], parts[1], dise, dis2e, s)

    return (emb_weight, s[:N])


# speedup vs baseline: 47.0230x; 1.1443x over previous
import functools

import jax
import jax.numpy as jnp
from jax import lax
from jax.experimental import pallas as pl
from jax.experimental.pallas import tpu as pltpu
from jax.experimental.pallas import tpu_sc as plsc

N = 50000
D = 32
NUM_LAYERS_K = 3
E = 1600000

NC = 2
NS = 16
NW = NC * NS
N_PAD = 50048
PAD_NODE = N
BLK = 128
SBLK = 256
CB = 14
EB_TILE = 196
NB_OUT = EB_TILE // CB
E_PAD = NW * EB_TILE * SBLK
RPT = N_PAD // NS
STG = 391
DEG_W = 16


def _zero_shared(buf2d, accum, row0, buf_rows):
    lanes = 32 if buf2d.dtype == jnp.bfloat16 else 16
    zvec = jnp.zeros((lanes,), buf2d.dtype)
    width = buf2d.shape[-1]
    npv = width // lanes

    def _zstore(i, _):
        for v in range(npv):
            buf2d[i, pl.ds(v * lanes, lanes)] = zvec
        return 0

    lax.fori_loop(0, buf_rows, _zstore, 0)

    def _zcopy(k, _):
        pltpu.sync_copy(buf2d, accum.at[pl.ds(row0 + k * buf_rows, buf_rows)])
        return 0

    nfull = RPT // buf_rows
    rem = RPT - nfull * buf_rows
    lax.fori_loop(0, nfull, _zcopy, 0)
    if rem:
        pltpu.sync_copy(
            buf2d.at[pl.ds(0, rem)],
            accum.at[pl.ds(row0 + nfull * buf_rows, rem)],
        )


def _spmv_body(p_hbm, frm_hbm, to_hbm, out_hbm, accum, table_sp, frm_v, to_v,
               rows_v, stage_v, sem0, sem1, ssem0, ssem1):
    cid = lax.axis_index("c")
    sid = lax.axis_index("s")
    wid = cid * NS + sid

    _zero_shared(rows_v.at[0], accum, sid * RPT, SBLK)

    nst = RPT // STG

    def _stage(k, _):
        r0 = sid * RPT + k * STG
        pltpu.sync_copy(p_hbm.at[pl.ds(r0, STG)], stage_v)
        pltpu.sync_copy(stage_v, table_sp.at[pl.ds(r0, STG)])
        return 0

    lax.fori_loop(0, nst, _stage, 0)
    plsc.subcore_barrier()

    gsems = (sem0, sem1)
    ssems = (ssem0, ssem1)
    nbatch = CB

    def _fire(b):
        s = b % 2
        return [
            pltpu.async_copy(
                table_sp.at[frm_v.at[b, pl.ds(i * 128, 128)]],
                rows_v.at[s, pl.ds(i * 128, 128)],
                gsems[s],
            )
            for i in range(2)
        ]

    def _outer(ob, _):
        r0 = wid * EB_TILE + ob * CB
        pltpu.sync_copy(frm_hbm.at[pl.ds(r0, CB)], frm_v)
        pltpu.sync_copy(to_hbm.at[pl.ds(r0, CB)], to_v)

        d_cur = _fire(0)
        pend = [None, None]
        for b in range(nbatch):
            s = b % 2
            if b + 1 < nbatch:
                if pend[1 - s] is not None:
                    for d in pend[1 - s]:
                        d.wait()
                    pend[1 - s] = None
                d_nxt = _fire(b + 1)
            else:
                d_nxt = None
            for d in d_cur:
                d.wait()
            pend[s] = [
                pltpu.async_copy(
                    rows_v.at[s],
                    accum.at[to_v.at[b]],
                    ssems[s],
                    add=True,
                )
            ]
            d_cur = d_nxt
        for p in pend:
            if p is not None:
                for d in p:
                    d.wait()
        return 0

    lax.fori_loop(0, NB_OUT, _outer, 0)
    plsc.subcore_barrier()

    pltpu.sync_copy(
        accum.at[pl.ds(sid * RPT, RPT)],
        out_hbm.at[cid, pl.ds(sid * RPT, RPT)],
    )


def _deg_body(to_hbm, out_hbm, accum, to_v, buf_v, dsem):
    cid = lax.axis_index("c")
    sid = lax.axis_index("s")
    wid = cid * NS + sid

    _zero_shared(buf_v, accum, sid * RPT, SBLK)
    plsc.subcore_barrier()

    onevec = jnp.ones((16,), jnp.float32)

    def _ostore(i, _):
        buf_v[i, pl.ds(0, 16)] = onevec
        return 0

    lax.fori_loop(0, SBLK, _ostore, 0)

    def _outer(ob, _):
        r0 = wid * EB_TILE + ob * CB
        pltpu.sync_copy(to_hbm.at[pl.ds(r0, CB)], to_v)
        descs = []
        for j in range(CB):
            descs.append(
                pltpu.async_copy(buf_v, accum.at[to_v.at[j]], dsem, add=True)
            )
        for d in descs:
            d.wait()
        return 0

    lax.fori_loop(0, NB_OUT, _outer, 0)
    plsc.subcore_barrier()

    pltpu.sync_copy(
        accum.at[pl.ds(sid * RPT, RPT)],
        out_hbm.at[cid, pl.ds(sid * RPT, RPT)],
    )


@functools.cache
def _get_sc_kernels():
    mesh = plsc.VectorSubcoreMesh(
        core_axis_name="c", subcore_axis_name="s", num_cores=NC, num_subcores=NS
    )
    spmv = pl.kernel(
        _spmv_body,
        out_type=jax.ShapeDtypeStruct((NC, N_PAD, D), jnp.bfloat16),
        mesh=mesh,
        scratch_types=[
            pltpu.VMEM_SHARED((N_PAD, D), jnp.bfloat16),
            pltpu.VMEM_SHARED((N_PAD, D), jnp.bfloat16),
            pltpu.VMEM((CB, SBLK), jnp.int32),
            pltpu.VMEM((CB, SBLK), jnp.int32),
            pltpu.VMEM((2, SBLK, D), jnp.bfloat16),
            pltpu.VMEM((STG, D), jnp.bfloat16),
            pltpu.SemaphoreType.DMA,
            pltpu.SemaphoreType.DMA,
            pltpu.SemaphoreType.DMA,
            pltpu.SemaphoreType.DMA,
        ],
        compiler_params=pltpu.CompilerParams(use_tc_tiling_on_sc=False),
    )
    deg = pl.kernel(
        _deg_body,
        out_type=jax.ShapeDtypeStruct((NC, N_PAD, DEG_W), jnp.float32),
        mesh=mesh,
        scratch_types=[
            pltpu.VMEM_SHARED((N_PAD, DEG_W), jnp.float32),
            pltpu.VMEM((CB, SBLK), jnp.int32),
            pltpu.VMEM((SBLK, DEG_W), jnp.float32),
            pltpu.SemaphoreType.DMA,
        ],
        compiler_params=pltpu.CompilerParams(use_tc_tiling_on_sc=False),
    )
    return spmv, deg


_TCB = N_PAD // 8


def _dis_body(a0, a1, emb, dise, dis2e, p0):
    deg = a0[...][:, 0:1] + a1[...][:, 0:1]
    d = jnp.where(deg > 0.0, lax.rsqrt(jnp.maximum(deg, 1e-30)), 0.0)
    de = jnp.broadcast_to(d, (_TCB, D))
    dise[...] = de
    dis2e[...] = de * de
    p0[...] = (de * emb[...]).astype(jnp.bfloat16)


_dis_tc = pl.pallas_call(
    _dis_body,
    grid=(8,),
    in_specs=[
        pl.BlockSpec((_TCB, DEG_W), lambda i: (i, 0)),
        pl.BlockSpec((_TCB, DEG_W), lambda i: (i, 0)),
        pl.BlockSpec((_TCB, D), lambda i: (i, 0)),
    ],
    out_specs=[pl.BlockSpec((_TCB, D), lambda i: (i, 0))] * 3,
    out_shape=[
        jax.ShapeDtypeStruct((N_PAD, D), jnp.float32),
        jax.ShapeDtypeStruct((N_PAD, D), jnp.float32),
        jax.ShapeDtypeStruct((N_PAD, D), jnp.bfloat16),
    ],
)


def _dense_body(scale, a0, a1, dise, dis2e, s_in, s_out, p_out):
    a = a0[...].astype(jnp.float32) + a1[...].astype(jnp.float32)
    s_out[...] = (s_in[...] + dise[...] * a) * scale
    p_out[...] = (dis2e[...] * a).astype(jnp.bfloat16)


def _make_dense(scale):
    return pl.pallas_call(
        functools.partial(_dense_body, scale),
        grid=(8,),
        in_specs=[pl.BlockSpec((_TCB, D), lambda i: (i, 0))] * 5,
        out_specs=[pl.BlockSpec((_TCB, D), lambda i: (i, 0))] * 2,
        out_shape=[
            jax.ShapeDtypeStruct((N_PAD, D), jnp.float32),
            jax.ShapeDtypeStruct((N_PAD, D), jnp.bfloat16),
        ],
    )


_dense_mid = _make_dense(1.0)
_dense_last = _make_dense(1.0 / (NUM_LAYERS_K + 1))


def kernel(emb_weight, edge_index):
    frm = edge_index[0].astype(jnp.int32)
    to = edge_index[1].astype(jnp.int32)
    pad = jnp.full((E_PAD - E,), PAD_NODE, jnp.int32)
    frm_p = jnp.concatenate([frm, pad]).reshape(E_PAD // SBLK, SBLK)
    to_p = jnp.concatenate([to, pad]).reshape(E_PAD // SBLK, SBLK)

    emb_pad = jnp.zeros((N_PAD, D), jnp.float32).at[:N].set(emb_weight)

    spmv, deg = _get_sc_kernels()
    g = deg(to_p)
    dise, dis2e, p = _dis_tc(g[0], g[1], emb_pad)

    s = emb_pad
    for layer in range(NUM_LAYERS_K):
        parts = spmv(p, frm_p, to_p)
        dense = _dense_last if layer == NUM_LAYERS_K - 1 else _dense_mid
        s, p = dense(parts[0], parts[1], dise, dis2e, s)

    return (emb_weight, s[:N])


# speedup vs baseline: 48.6605x; 1.0348x over previous
import functools

import jax
import jax.numpy as jnp
from jax import lax
from jax.experimental import pallas as pl
from jax.experimental.pallas import tpu as pltpu
from jax.experimental.pallas import tpu_sc as plsc

N = 50000
D = 32
NUM_LAYERS_K = 3
E = 1600000

NC = 2
NS = 16
NW = NC * NS
N_PAD = 50048
PAD_NODE = N
BLK = 128
SBLK = 256
CB = 14
EB_TILE = 196
NB_OUT = EB_TILE // CB
E_PAD = NW * EB_TILE * SBLK
RPT = N_PAD // NS
STG = 391
DEG_W = 16


def _zero_shared(buf2d, accum, row0, buf_rows):
    lanes = 32 if buf2d.dtype == jnp.bfloat16 else 16
    zvec = jnp.zeros((lanes,), buf2d.dtype)
    width = buf2d.shape[-1]
    npv = width // lanes

    def _zstore(i, _):
        for v in range(npv):
            buf2d[i, pl.ds(v * lanes, lanes)] = zvec
        return 0

    lax.fori_loop(0, buf_rows, _zstore, 0)

    def _zcopy(k, _):
        pltpu.sync_copy(buf2d, accum.at[pl.ds(row0 + k * buf_rows, buf_rows)])
        return 0

    nfull = RPT // buf_rows
    rem = RPT - nfull * buf_rows
    lax.fori_loop(0, nfull, _zcopy, 0)
    if rem:
        pltpu.sync_copy(
            buf2d.at[pl.ds(0, rem)],
            accum.at[pl.ds(row0 + nfull * buf_rows, rem)],
        )


def _spmv_body(p_hbm, frm_hbm, to_hbm, out_hbm, accum, table_sp, frm_v, to_v,
               rows_v, stage_v, sem0, sem1, ssem0, ssem1):
    cid = lax.axis_index("c")
    sid = lax.axis_index("s")
    wid = cid * NS + sid

    _zero_shared(rows_v.at[0], accum, sid * RPT, SBLK)

    nst = RPT // STG

    def _stage(k, _):
        r0 = sid * RPT + k * STG
        pltpu.sync_copy(p_hbm.at[pl.ds(r0, STG)], stage_v)
        pltpu.sync_copy(stage_v, table_sp.at[pl.ds(r0, STG)])
        return 0

    lax.fori_loop(0, nst, _stage, 0)
    plsc.subcore_barrier()

    gsems = (sem0, sem1)
    ssems = (ssem0, ssem1)
    nbatch = CB

    def _fire(b):
        s = b % 2
        return [
            pltpu.async_copy(
                table_sp.at[frm_v.at[b, pl.ds(i * 128, 128)]],
                rows_v.at[s, pl.ds(i * 128, 128)],
                gsems[s],
            )
            for i in range(2)
        ]

    def _outer(ob, _):
        r0 = wid * EB_TILE + ob * CB
        pltpu.sync_copy(frm_hbm.at[pl.ds(r0, CB)], frm_v)
        pltpu.sync_copy(to_hbm.at[pl.ds(r0, CB)], to_v)

        d_cur = _fire(0)
        pend = [None, None]
        for b in range(nbatch):
            s = b % 2
            if b + 1 < nbatch:
                if pend[1 - s] is not None:
                    for d in pend[1 - s]:
                        d.wait()
                    pend[1 - s] = None
                d_nxt = _fire(b + 1)
            else:
                d_nxt = None
            for d in d_cur:
                d.wait()
            pend[s] = [
                pltpu.async_copy(
                    rows_v.at[s],
                    accum.at[to_v.at[b]],
                    ssems[s],
                    add=True,
                )
            ]
            d_cur = d_nxt
        for p in pend:
            if p is not None:
                for d in p:
                    d.wait()
        return 0

    lax.fori_loop(0, NB_OUT, _outer, 0)
    plsc.subcore_barrier()

    pltpu.sync_copy(
        accum.at[pl.ds(sid * RPT, RPT)],
        out_hbm.at[cid, pl.ds(sid * RPT, RPT)],
    )


def _deg_body(to_hbm, out_hbm, accum, to_v, buf_v, dsem):
    cid = lax.axis_index("c")
    sid = lax.axis_index("s")
    wid = cid * NS + sid

    zpair = jnp.zeros((2, DEG_W), jnp.bfloat16)

    def _zstore(i, _):
        buf_v[pl.ds(2 * i, 2), :] = zpair
        return 0

    lax.fori_loop(0, SBLK // 2, _zstore, 0)

    def _zcopy(k, _):
        pltpu.sync_copy(buf_v, accum.at[pl.ds(sid * RPT + k * SBLK, SBLK)])
        return 0

    lax.fori_loop(0, RPT // SBLK, _zcopy, 0)
    zrem = RPT - (RPT // SBLK) * SBLK
    pltpu.sync_copy(
        buf_v.at[pl.ds(0, zrem)],
        accum.at[pl.ds(sid * RPT + (RPT // SBLK) * SBLK, zrem)],
    )
    plsc.subcore_barrier()

    opair = jnp.ones((2, DEG_W), jnp.bfloat16)

    def _ostore(i, _):
        buf_v[pl.ds(2 * i, 2), :] = opair
        return 0

    lax.fori_loop(0, SBLK // 2, _ostore, 0)

    def _outer(ob, _):
        r0 = wid * EB_TILE + ob * CB
        pltpu.sync_copy(to_hbm.at[pl.ds(r0, CB)], to_v)
        descs = []
        for j in range(CB):
            descs.append(
                pltpu.async_copy(buf_v, accum.at[to_v.at[j]], dsem, add=True)
            )
        for d in descs:
            d.wait()
        return 0

    lax.fori_loop(0, NB_OUT, _outer, 0)
    plsc.subcore_barrier()

    pltpu.sync_copy(
        accum.at[pl.ds(sid * RPT, RPT)],
        out_hbm.at[cid, pl.ds(sid * RPT, RPT)],
    )


@functools.cache
def _get_sc_kernels():
    mesh = plsc.VectorSubcoreMesh(
        core_axis_name="c", subcore_axis_name="s", num_cores=NC, num_subcores=NS
    )
    spmv = pl.kernel(
        _spmv_body,
        out_type=jax.ShapeDtypeStruct((NC, N_PAD, D), jnp.bfloat16),
        mesh=mesh,
        scratch_types=[
            pltpu.VMEM_SHARED((N_PAD, D), jnp.bfloat16),
            pltpu.VMEM_SHARED((N_PAD, D), jnp.bfloat16),
            pltpu.VMEM((CB, SBLK), jnp.int32),
            pltpu.VMEM((CB, SBLK), jnp.int32),
            pltpu.VMEM((2, SBLK, D), jnp.bfloat16),
            pltpu.VMEM((STG, D), jnp.bfloat16),
            pltpu.SemaphoreType.DMA,
            pltpu.SemaphoreType.DMA,
            pltpu.SemaphoreType.DMA,
            pltpu.SemaphoreType.DMA,
        ],
        compiler_params=pltpu.CompilerParams(use_tc_tiling_on_sc=False),
    )
    deg = pl.kernel(
        _deg_body,
        out_type=jax.ShapeDtypeStruct((NC, N_PAD, DEG_W), jnp.bfloat16),
        mesh=mesh,
        scratch_types=[
            pltpu.VMEM_SHARED((N_PAD, DEG_W), jnp.bfloat16),
            pltpu.VMEM((CB, SBLK), jnp.int32),
            pltpu.VMEM((SBLK, DEG_W), jnp.bfloat16),
            pltpu.SemaphoreType.DMA,
        ],
        compiler_params=pltpu.CompilerParams(use_tc_tiling_on_sc=False),
    )
    return spmv, deg


_TCB = N_PAD // 8


def _dis_body(a0, a1, emb, dise, dis2e, p0):
    deg = (a0[...][:, 0:1] + a1[...][:, 0:1]).astype(jnp.float32)
    d = jnp.where(deg > 0.0, lax.rsqrt(jnp.maximum(deg, 1e-30)), 0.0)
    de = jnp.broadcast_to(d, (_TCB, D))
    dise[...] = de
    dis2e[...] = de * de
    p0[...] = (de * emb[...]).astype(jnp.bfloat16)


_dis_tc = pl.pallas_call(
    _dis_body,
    grid=(8,),
    in_specs=[
        pl.BlockSpec((_TCB, DEG_W), lambda i: (i, 0)),
        pl.BlockSpec((_TCB, DEG_W), lambda i: (i, 0)),
        pl.BlockSpec((_TCB, D), lambda i: (i, 0)),
    ],
    out_specs=[pl.BlockSpec((_TCB, D), lambda i: (i, 0))] * 3,
    out_shape=[
        jax.ShapeDtypeStruct((N_PAD, D), jnp.float32),
        jax.ShapeDtypeStruct((N_PAD, D), jnp.float32),
        jax.ShapeDtypeStruct((N_PAD, D), jnp.bfloat16),
    ],
)


def _dense_body(scale, a0, a1, dise, dis2e, s_in, s_out, p_out):
    a = a0[...].astype(jnp.float32) + a1[...].astype(jnp.float32)
    s_out[...] = (s_in[...] + dise[...] * a) * scale
    p_out[...] = (dis2e[...] * a).astype(jnp.bfloat16)


def _make_dense(scale):
    return pl.pallas_call(
        functools.partial(_dense_body, scale),
        grid=(8,),
        in_specs=[pl.BlockSpec((_TCB, D), lambda i: (i, 0))] * 5,
        out_specs=[pl.BlockSpec((_TCB, D), lambda i: (i, 0))] * 2,
        out_shape=[
            jax.ShapeDtypeStruct((N_PAD, D), jnp.float32),
            jax.ShapeDtypeStruct((N_PAD, D), jnp.bfloat16),
        ],
    )


_dense_mid = _make_dense(1.0)
_dense_last = _make_dense(1.0 / (NUM_LAYERS_K + 1))


def kernel(emb_weight, edge_index):
    frm = edge_index[0].astype(jnp.int32)
    to = edge_index[1].astype(jnp.int32)
    pad = jnp.full((E_PAD - E,), PAD_NODE, jnp.int32)
    frm_p = jnp.concatenate([frm, pad]).reshape(E_PAD // SBLK, SBLK)
    to_p = jnp.concatenate([to, pad]).reshape(E_PAD // SBLK, SBLK)

    emb_pad = jnp.zeros((N_PAD, D), jnp.float32).at[:N].set(emb_weight)

    spmv, deg = _get_sc_kernels()
    g = deg(to_p)
    dise, dis2e, p = _dis_tc(g[0], g[1], emb_pad)

    s = emb_pad
    for layer in range(NUM_LAYERS_K):
        parts = spmv(p, frm_p, to_p)
        dense = _dense_last if layer == NUM_LAYERS_K - 1 else _dense_mid
        s, p = dense(parts[0], parts[1], dise, dis2e, s)

    return (emb_weight, s[:N])
